# baseline (speedup 1.0000x reference)
import jax
import jax.numpy as jnp
from jax import lax
from jax.experimental import pallas as pl
from jax.experimental.pallas import tpu as pltpu
from jax.experimental.pallas import tpu_sc as plsc

N = 10000
E = 320000
D_IN = 128
HID = 128
C = 64
K = 10
ALPHA = 0.1

NP_ = 10240
NTILES = 32
NBLK = 80
BLK = 128
NBUF = 4
EPT = NBLK * BLK
EP = NTILES * EPT
ROWS_PER_TILE = NP_ // 16
CHUNK = 64


def _sc_mesh():
    return plsc.VectorSubcoreMesh(core_axis_name="c", subcore_axis_name="s")


_SC_PARAMS = pltpu.CompilerParams(use_tc_tiling_on_sc=False)


def _deg_body(colp, out, acc, colv, ones, bounce, sem):
    c = lax.axis_index("c")
    s = lax.axis_index("s")
    wid = c * 16 + s
    pltpu.sync_copy(colp.at[wid], colv)
    for i in range(BLK // 16):
        ones[pl.ds(i * 16, 16)] = jnp.ones((16,), jnp.float32)
    for i in range(ROWS_PER_TILE // 16):
        bounce[pl.ds(i * 16, 16)] = jnp.zeros((16,), jnp.float32)
    pltpu.sync_copy(bounce, acc.at[pl.ds(s * ROWS_PER_TILE, ROWS_PER_TILE)])
    plsc.subcore_barrier()

    @pl.loop(0, NBLK)
    def _blk(b):
        pltpu.async_copy(ones, acc.at[colv.at[b]], sem, add=True)

    @pl.loop(0, NBLK)
    def _drain(b):
        pltpu.make_async_copy(ones, acc.at[colv.at[b]], sem).wait()

    plsc.subcore_barrier()
    pltpu.sync_copy(acc.at[pl.ds(s * ROWS_PER_TILE, ROWS_PER_TILE)], bounce)
    pltpu.sync_copy(bounce, out.at[c, pl.ds(s * ROWS_PER_TILE, ROWS_PER_TILE)])


def _deg_kernel(colp):
    f = pl.kernel(
        _deg_body,
        out_type=jax.ShapeDtypeStruct((2, NP_), jnp.float32),
        mesh=_sc_mesh(),
        scratch_types=[
            pltpu.VMEM_SHARED((NP_,), jnp.float32),
            pltpu.VMEM((NBLK, BLK), jnp.int32),
            pltpu.VMEM((BLK,), jnp.float32),
            pltpu.VMEM((ROWS_PER_TILE,), jnp.float32),
            pltpu.SemaphoreType.DMA,
        ],
        compiler_params=_SC_PARAMS,
    )
    return f(colp)


def _zero_zb(zb):
    for i in range(CHUNK):
        for j in range(C // 16):
            zb[i, pl.ds(j * 16, 16)] = jnp.zeros((16,), jnp.float32)


def _stream_phase(g, acc, rowv, colv, gbuf, gsem, ssem):
    for b in range(NBUF):
        pltpu.async_copy(g.at[rowv.at[b]], gbuf.at[b], gsem.at[b])

    @pl.loop(0, NBLK // NBUF)
    def _blk(i):
        descs = []
        for sl in range(NBUF):
            b = i * NBUF + sl
            pltpu.make_async_copy(g.at[rowv.at[b]], gbuf.at[sl],
                                  gsem.at[sl]).wait()
            descs.append(pltpu.async_copy(gbuf.at[sl], acc.at[colv.at[b]],
                                          ssem.at[sl], add=True))
        for sl in range(NBUF):
            descs[sl].wait()
            b2 = i * NBUF + sl + NBUF

            @pl.when(b2 < NBLK)
            def _():
                pltpu.async_copy(g.at[rowv.at[b2]], gbuf.at[sl], gsem.at[sl])


def _dump_partials(acc, out, zb, c, s):
    for t in range(ROWS_PER_TILE // CHUNK):
        base = s * ROWS_PER_TILE + t * CHUNK
        pltpu.sync_copy(acc.at[pl.ds(base, CHUNK)], zb)
        pltpu.sync_copy(zb, out.at[c, pl.ds(base, CHUNK)])


def _scat_body(g, rowp, colp, out, acc, rowv, colv, gbuf, zb, gsem, ssem):
    c = lax.axis_index("c")
    s = lax.axis_index("s")
    wid = c * 16 + s
    pltpu.sync_copy(rowp.at[wid], rowv)
    pltpu.sync_copy(colp.at[wid], colv)
    _zero_zb(zb)
    for t in range(ROWS_PER_TILE // CHUNK):
        pltpu.sync_copy(zb, acc.at[pl.ds(s * ROWS_PER_TILE + t * CHUNK, CHUNK)])
    plsc.subcore_barrier()
    _stream_phase(g, acc, rowv, colv, gbuf, gsem, ssem)
    plsc.subcore_barrier()
    _dump_partials(acc, out, zb, c, s)


def _scat_kernel(g, rowp, colp):
    f = pl.kernel(
        _scat_body,
        out_type=jax.ShapeDtypeStruct((2, NP_, C), jnp.float32),
        mesh=_sc_mesh(),
        scratch_types=[
            pltpu.VMEM_SHARED((NP_, C), jnp.float32),
            pltpu.VMEM((NBLK, BLK), jnp.int32),
            pltpu.VMEM((NBLK, BLK), jnp.int32),
            pltpu.VMEM((NBUF, BLK, C), jnp.float32),
            pltpu.VMEM((CHUNK, C), jnp.float32),
            pltpu.SemaphoreType.DMA((NBUF,)),
            pltpu.SemaphoreType.DMA((NBUF,)),
        ],
        compiler_params=_SC_PARAMS,
    )
    return f(g, rowp, colp)


def _fused_body(p, gprev, g0, c1f, rowp, colp, pout, gout,
                acc, rowv, colv, gbuf, zb, ab, b0, b1, b2, gsem, ssem):
    c = lax.axis_index("c")
    s = lax.axis_index("s")
    wid = c * 16 + s
    pltpu.sync_copy(rowp.at[wid], rowv)
    pltpu.sync_copy(colp.at[wid], colv)
    _zero_zb(zb)
    for t in range(ROWS_PER_TILE // CHUNK):
        base = s * ROWS_PER_TILE + t * CHUNK
        pltpu.sync_copy(p.at[0, pl.ds(base, CHUNK)], b0)
        pltpu.sync_copy(p.at[1, pl.ds(base, CHUNK)], b1)
        pltpu.sync_copy(gprev.at[pl.ds(base, CHUNK)], b2)
        pltpu.sync_copy(c1f.at[pl.ds(base, CHUNK)], ab)

        @pl.loop(0, CHUNK)
        def _mul(r):
            for j in range(C // 16):
                sl = pl.ds(j * 16, 16)
                ab[r, sl] = ab[r, sl] * (b0[r, sl] + b1[r, sl] + b2[r, sl])

        pltpu.sync_copy(g0.at[pl.ds(base, CHUNK)], b0)

        @pl.loop(0, CHUNK)
        def _tele(r):
            for j in range(C // 16):
                sl = pl.ds(j * 16, 16)
                ab[r, sl] = ab[r, sl] + ALPHA * b0[r, sl]

        pltpu.sync_copy(ab, gout.at[pl.ds(base, CHUNK)])
        pltpu.sync_copy(zb, acc.at[pl.ds(base, CHUNK)])
    plsc.subcore_barrier()
    _stream_phase(gout, acc, rowv, colv, gbuf, gsem, ssem)
    plsc.subcore_barrier()
    _dump_partials(acc, pout, zb, c, s)


def _fused_kernel(p, gprev, g0, c1f, rowp, colp):
    f = pl.kernel(
        _fused_body,
        out_type=(
            jax.ShapeDtypeStruct((2, NP_, C), jnp.float32),
            jax.ShapeDtypeStruct((NP_, C), jnp.float32),
        ),
        mesh=_sc_mesh(),
        scratch_types=[
            pltpu.VMEM_SHARED((NP_, C), jnp.float32),
            pltpu.VMEM((NBLK, BLK), jnp.int32),
            pltpu.VMEM((NBLK, BLK), jnp.int32),
            pltpu.VMEM((NBUF, BLK, C), jnp.float32),
            pltpu.VMEM((CHUNK, C), jnp.float32),
            pltpu.VMEM((CHUNK, C), jnp.float32),
            pltpu.VMEM((CHUNK, C), jnp.float32),
            pltpu.VMEM((CHUNK, C), jnp.float32),
            pltpu.VMEM((CHUNK, C), jnp.float32),
            pltpu.SemaphoreType.DMA((NBUF,)),
            pltpu.SemaphoreType.DMA((NBUF,)),
        ],
        compiler_params=_SC_PARAMS,
    )
    return f(p, gprev, g0, c1f, rowp, colp)


def _mlp_body(x_ref, w1_ref, b1_ref, w2_ref, b2_ref, d0_ref, d1_ref,
              g0_ref, c1_ref, deg_ref):
    h1 = jnp.maximum(
        jnp.dot(x_ref[...], w1_ref[...], preferred_element_type=jnp.float32)
        + b1_ref[...], 0.0)
    h = jnp.dot(h1, w2_ref[...], preferred_element_type=jnp.float32) + b2_ref[...]
    deg = d0_ref[...] + d1_ref[...] + 1.0
    dinv = lax.rsqrt(deg)
    g0_ref[...] = h * dinv
    c1_ref[...] = ((1.0 - ALPHA) / deg) + jnp.zeros((1, C), jnp.float32)
    deg_ref[...] = deg


def _mlp_kernel(xp, W1, b1, W2, b2, d0, d1):
    bm = 1024
    grid = NP_ // bm
    return pl.pallas_call(
        _mlp_body,
        grid=(grid,),
        in_specs=[
            pl.BlockSpec((bm, D_IN), lambda i: (i, 0)),
            pl.BlockSpec((D_IN, HID), lambda i: (0, 0)),
            pl.BlockSpec((1, HID), lambda i: (0, 0)),
            pl.BlockSpec((HID, C), lambda i: (0, 0)),
            pl.BlockSpec((1, C), lambda i: (0, 0)),
            pl.BlockSpec((bm, 1), lambda i: (i, 0)),
            pl.BlockSpec((bm, 1), lambda i: (i, 0)),
        ],
        out_specs=[
            pl.BlockSpec((bm, C), lambda i: (i, 0)),
            pl.BlockSpec((bm, C), lambda i: (i, 0)),
            pl.BlockSpec((bm, 1), lambda i: (i, 0)),
        ],
        out_shape=[
            jax.ShapeDtypeStruct((NP_, C), jnp.float32),
            jax.ShapeDtypeStruct((NP_, C), jnp.float32),
            jax.ShapeDtypeStruct((NP_, 1), jnp.float32),
        ],
    )(xp, W1, b1, W2, b2, d0, d1)


def _final_body(p0_ref, p1_ref, g_ref, g0_ref, c1_ref, deg_ref, out_ref):
    gk = c1_ref[...] * (p0_ref[...] + p1_ref[...] + g_ref[...]) \
        + ALPHA * g0_ref[...]
    h = gk * jnp.sqrt(deg_ref[...])
    m = jnp.max(h, axis=1, keepdims=True)
    e = jnp.exp(h - m)
    ssum = jnp.sum(e, axis=1, keepdims=True)
    out_ref[...] = h - m - jnp.log(ssum)


def _final_kernel(p0, p1, g, g0, c1, deg):
    bm = 1000
    grid = N // bm
    row = pl.BlockSpec((bm, C), lambda i: (i, 0))
    return pl.pallas_call(
        _final_body,
        grid=(grid,),
        in_specs=[row, row, row, row, row,
                  pl.BlockSpec((bm, 1), lambda i: (i, 0))],
        out_specs=row,
        out_shape=jax.ShapeDtypeStruct((N, C), jnp.float32),
    )(p0, p1, g, g0, c1, deg)


def kernel(x, edge_index, W1, b1, W2, b2):
    row = edge_index[0]
    col = edge_index[1]
    pad = EP - E
    pad_src = (jnp.arange(pad, dtype=jnp.int32) * 37) % N
    pad_dst = N + (jnp.arange(pad, dtype=jnp.int32) % (NP_ - N))
    rowp = jnp.concatenate([row, pad_src]).reshape(NTILES, NBLK, BLK)
    colp = jnp.concatenate([col, pad_dst]).reshape(NTILES, NBLK, BLK)

    dpart = _deg_kernel(colp)
    d0 = dpart[0].reshape(NP_, 1)
    d1 = dpart[1].reshape(NP_, 1)

    xp = jnp.pad(x, ((0, NP_ - N), (0, 0)))
    g0, c1, deg = _mlp_kernel(xp, W1, b1.reshape(1, HID), W2,
                              b2.reshape(1, C), d0, d1)

    g = g0
    p = _scat_kernel(g, rowp, colp)
    for _ in range(K - 1):
        p, g = _fused_kernel(p, g, g0, c1, rowp, colp)

    return _final_kernel(p[0][:N], p[1][:N], g[:N], g0[:N], c1[:N], deg[:N])


# speedup vs baseline: 1.2926x; 1.2926x over previous
import jax
import jax.numpy as jnp
from jax import lax
from jax.experimental import pallas as pl
from jax.experimental.pallas import tpu as pltpu
from jax.experimental.pallas import tpu_sc as plsc

N = 10000
E = 320000
D_IN = 128
HID = 128
C = 64
K = 10
ALPHA = 0.1

NP_ = 10240
NTILES = 32
NBLK = 80
BLK = 128
NBUF = 4
EPT = NBLK * BLK
EP = NTILES * EPT
ROWS_PER_TILE = NP_ // 16
CHUNK = 64


def _sc_mesh():
    return plsc.VectorSubcoreMesh(core_axis_name="c", subcore_axis_name="s")


_SC_PARAMS = pltpu.CompilerParams(use_tc_tiling_on_sc=False)


def _deg_body(colp, out, acc, colv, ones, bounce, sem):
    c = lax.axis_index("c")
    s = lax.axis_index("s")
    wid = c * 16 + s
    pltpu.sync_copy(colp.at[wid], colv)
    for i in range(BLK // 16):
        ones[pl.ds(i * 16, 16)] = jnp.ones((16,), jnp.float32)
    for i in range(ROWS_PER_TILE // 16):
        bounce[pl.ds(i * 16, 16)] = jnp.zeros((16,), jnp.float32)
    pltpu.sync_copy(bounce, acc.at[pl.ds(s * ROWS_PER_TILE, ROWS_PER_TILE)])
    plsc.subcore_barrier()

    @pl.loop(0, NBLK)
    def _blk(b):
        pltpu.async_copy(ones, acc.at[colv.at[b]], sem, add=True)

    @pl.loop(0, NBLK)
    def _drain(b):
        pltpu.make_async_copy(ones, acc.at[colv.at[b]], sem).wait()

    plsc.subcore_barrier()
    pltpu.sync_copy(acc.at[pl.ds(s * ROWS_PER_TILE, ROWS_PER_TILE)], bounce)
    pltpu.sync_copy(bounce, out.at[c, pl.ds(s * ROWS_PER_TILE, ROWS_PER_TILE)])


def _deg_kernel(colp):
    f = pl.kernel(
        _deg_body,
        out_type=jax.ShapeDtypeStruct((2, NP_), jnp.float32),
        mesh=_sc_mesh(),
        scratch_types=[
            pltpu.VMEM_SHARED((NP_,), jnp.float32),
            pltpu.VMEM((NBLK, BLK), jnp.int32),
            pltpu.VMEM((BLK,), jnp.float32),
            pltpu.VMEM((ROWS_PER_TILE,), jnp.float32),
            pltpu.SemaphoreType.DMA,
        ],
        compiler_params=_SC_PARAMS,
    )
    return f(colp)


def _zero_zb(zb):
    for i in range(CHUNK):
        for j in range(C // 16):
            zb[i, pl.ds(j * 16, 16)] = jnp.zeros((16,), jnp.float32)


def _stream_phase(g, acc, rowv, colv, gbuf, gsem, ssem):
    for b in range(NBUF):
        pltpu.async_copy(g.at[rowv.at[b]], gbuf.at[b], gsem.at[b])

    @pl.loop(0, NBLK // NBUF)
    def _blk(i):
        descs = []
        for sl in range(NBUF):
            b = i * NBUF + sl
            pltpu.make_async_copy(g.at[rowv.at[b]], gbuf.at[sl],
                                  gsem.at[sl]).wait()
            descs.append(pltpu.async_copy(gbuf.at[sl], acc.at[colv.at[b]],
                                          ssem.at[sl], add=True))
        for sl in range(NBUF):
            descs[sl].wait()
            b2 = i * NBUF + sl + NBUF

            @pl.when(b2 < NBLK)
            def _():
                pltpu.async_copy(g.at[rowv.at[b2]], gbuf.at[sl], gsem.at[sl])


def _dump_partials(acc, out, zb, c, s):
    for t in range(ROWS_PER_TILE // CHUNK):
        base = s * ROWS_PER_TILE + t * CHUNK
        pltpu.sync_copy(acc.at[pl.ds(base, CHUNK)], zb)
        pltpu.sync_copy(zb, out.at[c, pl.ds(base, CHUNK)])


def _scat_body(g, rowp, colp, out, acc, rowv, colv, gbuf, zb, gsem, ssem):
    c = lax.axis_index("c")
    s = lax.axis_index("s")
    wid = c * 16 + s
    pltpu.sync_copy(rowp.at[wid], rowv)
    pltpu.sync_copy(colp.at[wid], colv)
    _zero_zb(zb)
    for t in range(ROWS_PER_TILE // CHUNK):
        pltpu.sync_copy(zb, acc.at[pl.ds(s * ROWS_PER_TILE + t * CHUNK, CHUNK)])
    plsc.subcore_barrier()
    _stream_phase(g, acc, rowv, colv, gbuf, gsem, ssem)
    plsc.subcore_barrier()
    _dump_partials(acc, out, zb, c, s)


def _scat_kernel(g, rowp, colp):
    f = pl.kernel(
        _scat_body,
        out_type=jax.ShapeDtypeStruct((2, NP_, C), jnp.float32),
        mesh=_sc_mesh(),
        scratch_types=[
            pltpu.VMEM_SHARED((NP_, C), jnp.float32),
            pltpu.VMEM((NBLK, BLK), jnp.int32),
            pltpu.VMEM((NBLK, BLK), jnp.int32),
            pltpu.VMEM((NBUF, BLK, C), jnp.float32),
            pltpu.VMEM((CHUNK, C), jnp.float32),
            pltpu.SemaphoreType.DMA((NBUF,)),
            pltpu.SemaphoreType.DMA((NBUF,)),
        ],
        compiler_params=_SC_PARAMS,
    )
    return f(g, rowp, colp)


def _fused_body(p, gprev, g0, c1f, rowp, colp, pout, gout,
                acc, rowv, colv, gbuf, zb, ab, ab2, b0, b1, b2, b3,
                gsem, ssem, wsem):
    c = lax.axis_index("c")
    s = lax.axis_index("s")
    wid = c * 16 + s
    pltpu.sync_copy(rowp.at[wid], rowv)
    pltpu.sync_copy(colp.at[wid], colv)
    _zero_zb(zb)
    wouts = {}
    drains = []
    for t in range(ROWS_PER_TILE // CHUNK):
        base = s * ROWS_PER_TILE + t * CHUNK
        out = ab if t % 2 == 0 else ab2
        if t >= 2:
            wouts[t - 2].wait()
        stages = [
            pltpu.async_copy(p.at[0, pl.ds(base, CHUNK)], b0, gsem.at[0]),
            pltpu.async_copy(p.at[1, pl.ds(base, CHUNK)], b1, gsem.at[1]),
            pltpu.async_copy(gprev.at[pl.ds(base, CHUNK)], b2, gsem.at[2]),
            pltpu.async_copy(g0.at[pl.ds(base, CHUNK)], b3, gsem.at[3]),
            pltpu.async_copy(c1f.at[pl.ds(base, CHUNK)], out, ssem.at[0]),
        ]
        for d in stages:
            d.wait()

        @pl.loop(0, CHUNK)
        def _row(r):
            for j in range(C // 16):
                sl = pl.ds(j * 16, 16)
                out[r, sl] = out[r, sl] * (b0[r, sl] + b1[r, sl] + b2[r, sl]) \
                    + ALPHA * b3[r, sl]

        wouts[t] = pltpu.async_copy(out, gout.at[pl.ds(base, CHUNK)], wsem)
        drains.append(pltpu.async_copy(zb, acc.at[pl.ds(base, CHUNK)],
                                       ssem.at[1]))
    for t in (ROWS_PER_TILE // CHUNK - 2, ROWS_PER_TILE // CHUNK - 1):
        wouts[t].wait()
    for d in drains:
        d.wait()
    plsc.subcore_barrier()
    _stream_phase(gout, acc, rowv, colv, gbuf, gsem, ssem)
    plsc.subcore_barrier()
    _dump_partials(acc, pout, zb, c, s)


def _fused_kernel(p, gprev, g0, c1f, rowp, colp):
    f = pl.kernel(
        _fused_body,
        out_type=(
            jax.ShapeDtypeStruct((2, NP_, C), jnp.float32),
            jax.ShapeDtypeStruct((NP_, C), jnp.float32),
        ),
        mesh=_sc_mesh(),
        scratch_types=[
            pltpu.VMEM_SHARED((NP_, C), jnp.float32),
            pltpu.VMEM((NBLK, BLK), jnp.int32),
            pltpu.VMEM((NBLK, BLK), jnp.int32),
            pltpu.VMEM((NBUF, BLK, C), jnp.float32),
            pltpu.VMEM((CHUNK, C), jnp.float32),
            pltpu.VMEM((CHUNK, C), jnp.float32),
            pltpu.VMEM((CHUNK, C), jnp.float32),
            pltpu.VMEM((CHUNK, C), jnp.float32),
            pltpu.VMEM((CHUNK, C), jnp.float32),
            pltpu.VMEM((CHUNK, C), jnp.float32),
            pltpu.VMEM((CHUNK, C), jnp.float32),
            pltpu.SemaphoreType.DMA((NBUF,)),
            pltpu.SemaphoreType.DMA((NBUF,)),
            pltpu.SemaphoreType.DMA,
        ],
        compiler_params=_SC_PARAMS,
    )
    return f(p, gprev, g0, c1f, rowp, colp)


def _mlp_body(x_ref, w1_ref, b1_ref, w2_ref, b2_ref, d0_ref, d1_ref,
              g0_ref, c1_ref, deg_ref):
    h1 = jnp.maximum(
        jnp.dot(x_ref[...], w1_ref[...], preferred_element_type=jnp.float32)
        + b1_ref[...], 0.0)
    h = jnp.dot(h1, w2_ref[...], preferred_element_type=jnp.float32) + b2_ref[...]
    deg = d0_ref[...] + d1_ref[...] + 1.0
    dinv = lax.rsqrt(deg)
    g0_ref[...] = h * dinv
    c1_ref[...] = ((1.0 - ALPHA) / deg) + jnp.zeros((1, C), jnp.float32)
    deg_ref[...] = deg


def _mlp_kernel(xp, W1, b1, W2, b2, d0, d1):
    bm = 1024
    grid = NP_ // bm
    return pl.pallas_call(
        _mlp_body,
        grid=(grid,),
        in_specs=[
            pl.BlockSpec((bm, D_IN), lambda i: (i, 0)),
            pl.BlockSpec((D_IN, HID), lambda i: (0, 0)),
            pl.BlockSpec((1, HID), lambda i: (0, 0)),
            pl.BlockSpec((HID, C), lambda i: (0, 0)),
            pl.BlockSpec((1, C), lambda i: (0, 0)),
            pl.BlockSpec((bm, 1), lambda i: (i, 0)),
            pl.BlockSpec((bm, 1), lambda i: (i, 0)),
        ],
        out_specs=[
            pl.BlockSpec((bm, C), lambda i: (i, 0)),
            pl.BlockSpec((bm, C), lambda i: (i, 0)),
            pl.BlockSpec((bm, 1), lambda i: (i, 0)),
        ],
        out_shape=[
            jax.ShapeDtypeStruct((NP_, C), jnp.float32),
            jax.ShapeDtypeStruct((NP_, C), jnp.float32),
            jax.ShapeDtypeStruct((NP_, 1), jnp.float32),
        ],
    )(xp, W1, b1, W2, b2, d0, d1)


def _final_body(p0_ref, p1_ref, g_ref, g0_ref, c1_ref, deg_ref, out_ref):
    gk = c1_ref[...] * (p0_ref[...] + p1_ref[...] + g_ref[...]) \
        + ALPHA * g0_ref[...]
    h = gk * jnp.sqrt(deg_ref[...])
    m = jnp.max(h, axis=1, keepdims=True)
    e = jnp.exp(h - m)
    ssum = jnp.sum(e, axis=1, keepdims=True)
    out_ref[...] = h - m - jnp.log(ssum)


def _final_kernel(p0, p1, g, g0, c1, deg):
    bm = 1000
    grid = N // bm
    row = pl.BlockSpec((bm, C), lambda i: (i, 0))
    return pl.pallas_call(
        _final_body,
        grid=(grid,),
        in_specs=[row, row, row, row, row,
                  pl.BlockSpec((bm, 1), lambda i: (i, 0))],
        out_specs=row,
        out_shape=jax.ShapeDtypeStruct((N, C), jnp.float32),
    )(p0, p1, g, g0, c1, deg)


def kernel(x, edge_index, W1, b1, W2, b2):
    row = edge_index[0]
    col = edge_index[1]
    pad = EP - E
    pad_src = (jnp.arange(pad, dtype=jnp.int32) * 37) % N
    pad_dst = N + (jnp.arange(pad, dtype=jnp.int32) % (NP_ - N))
    rowp = jnp.concatenate([row, pad_src]).reshape(NTILES, NBLK, BLK)
    colp = jnp.concatenate([col, pad_dst]).reshape(NTILES, NBLK, BLK)

    dpart = _deg_kernel(colp)
    d0 = dpart[0].reshape(NP_, 1)
    d1 = dpart[1].reshape(NP_, 1)

    xp = jnp.pad(x, ((0, NP_ - N), (0, 0)))
    g0, c1, deg = _mlp_kernel(xp, W1, b1.reshape(1, HID), W2,
                              b2.reshape(1, C), d0, d1)

    g = g0
    p = _scat_kernel(g, rowp, colp)
    for _ in range(K - 1):
        p, g = _fused_kernel(p, g, g0, c1, rowp, colp)

    return _final_kernel(p[0][:N], p[1][:N], g[:N], g0[:N], c1[:N], deg[:N])
